# baseline (speedup 1.0000x reference)
import functools

import jax
import jax.numpy as jnp
from jax.experimental import pallas as pl
from jax.experimental.pallas import tpu as pltpu

F32 = jnp.float32


def _prep_kernel(idx_ref, ew_ref, c_ref, g_ref, u_ref, m_ref):
    idx = idx_ref[0]
    g = g_ref[idx]
    c = c_ref[...]
    ew = ew_ref[...]
    w = jnp.dot(ew, c, preferred_element_type=F32)
    p_dim, r, _ = g.shape
    m0 = jnp.zeros((r, r), dtype=F32)
    for p in range(p_dim):
        onehot = (jax.lax.broadcasted_iota(jnp.int32, (p_dim, 1), 0) == p)
        wp = jnp.dot(w, onehot.astype(F32), preferred_element_type=F32)
        m0 = m0 + wp * g[p]
    gn2 = jnp.sum(g * g)
    cn2 = jnp.sum(c * c)
    un2 = jnp.sum(u_ref[...] * u_ref[...])
    scale = jax.lax.rsqrt(gn2) * jax.lax.rsqrt(cn2) / un2
    m_ref[...] = m0 * scale


def _p2_kernel(x_ref, u_ref, m_ref, o_ref):
    xu = jnp.dot(x_ref[...], u_ref[...], preferred_element_type=F32)
    o_ref[...] = jnp.dot(xu, m_ref[...], preferred_element_type=F32)


def _main_kernel(x_ref, w_ref, p2_ref, u_ref, b_ref, o_ref):
    acc = jnp.dot(x_ref[...], w_ref[...], preferred_element_type=F32)
    adapt = jax.lax.dot_general(
        p2_ref[...], u_ref[...], (((1,), (1,)), ((), ())),
        preferred_element_type=F32)
    o_ref[...] = acc + adapt + b_ref[...]


@functools.partial(jax.jit, static_argnames=())
def kernel(x, tensor_idx, expert_weights, W, b, G, C, U):
    n, d_in = x.shape
    d_out = W.shape[1]
    k_dim, p_dim, r, _ = G.shape
    t_dim = expert_weights.shape[0]

    idx = jnp.asarray(tensor_idx, jnp.int32).reshape((1,))
    ew2 = expert_weights.reshape(1, t_dim).astype(F32)

    m_eff = pl.pallas_call(
        _prep_kernel,
        grid_spec=pltpu.PrefetchScalarGridSpec(
            num_scalar_prefetch=1,
            grid=(1,),
            in_specs=[
                pl.BlockSpec((1, t_dim), lambda i, idx_ref: (0, 0)),
                pl.BlockSpec((t_dim, p_dim), lambda i, idx_ref: (0, 0)),
                pl.BlockSpec((k_dim, p_dim, r, r), lambda i, idx_ref: (0, 0, 0, 0)),
                pl.BlockSpec((d_in, r), lambda i, idx_ref: (0, 0)),
            ],
            out_specs=pl.BlockSpec((r, r), lambda i, idx_ref: (0, 0)),
        ),
        out_shape=jax.ShapeDtypeStruct((r, r), F32),
    )(idx, ew2, C, G, U)

    bn2 = 1024
    p2 = pl.pallas_call(
        _p2_kernel,
        grid=(n // bn2,),
        in_specs=[
            pl.BlockSpec((bn2, d_in), lambda i: (i, 0)),
            pl.BlockSpec((d_in, r), lambda i: (0, 0)),
            pl.BlockSpec((r, r), lambda i: (0, 0)),
        ],
        out_specs=pl.BlockSpec((bn2, r), lambda i: (i, 0)),
        out_shape=jax.ShapeDtypeStruct((n, r), F32),
        compiler_params=pltpu.CompilerParams(
            dimension_semantics=("parallel",)),
    )(x, U, m_eff)

    bn, bj = 512, 512
    b2 = b.reshape(1, d_out)
    out = pl.pallas_call(
        _main_kernel,
        grid=(n // bn, d_out // bj),
        in_specs=[
            pl.BlockSpec((bn, d_in), lambda i, j: (i, 0)),
            pl.BlockSpec((d_in, bj), lambda i, j: (0, j)),
            pl.BlockSpec((bn, r), lambda i, j: (i, 0)),
            pl.BlockSpec((bj, r), lambda i, j: (j, 0)),
            pl.BlockSpec((1, bj), lambda i, j: (0, j)),
        ],
        out_specs=pl.BlockSpec((bn, bj), lambda i, j: (i, j)),
        out_shape=jax.ShapeDtypeStruct((n, d_out), F32),
        compiler_params=pltpu.CompilerParams(
            dimension_semantics=("parallel", "parallel")),
    )(x, W, p2, U, b2)
    return out


# speedup vs baseline: 1.0800x; 1.0800x over previous
import functools

import jax
import jax.numpy as jnp
from jax.experimental import pallas as pl
from jax.experimental.pallas import tpu as pltpu

F32 = jnp.float32


def _prep_kernel(idx_ref, ew_ref, c_ref, g_ref, u_ref, m_ref):
    idx = idx_ref[0]
    g = g_ref[idx]
    c = c_ref[...]
    ew = ew_ref[...]
    w = jnp.dot(ew, c, preferred_element_type=F32)
    p_dim, r, _ = g.shape
    m0 = jnp.zeros((r, r), dtype=F32)
    for p in range(p_dim):
        onehot = (jax.lax.broadcasted_iota(jnp.int32, (p_dim, 1), 0) == p)
        wp = jnp.dot(w, onehot.astype(F32), preferred_element_type=F32)
        m0 = m0 + wp * g[p]
    gn2 = jnp.sum(g * g)
    cn2 = jnp.sum(c * c)
    un2 = jnp.sum(u_ref[...] * u_ref[...])
    scale = jax.lax.rsqrt(gn2) * jax.lax.rsqrt(cn2) / un2
    m_ref[...] = m0 * scale


def _p2_kernel(x_ref, u_ref, m_ref, o_ref):
    xu = jnp.dot(x_ref[...], u_ref[...], preferred_element_type=F32)
    o_ref[...] = jnp.dot(xu, m_ref[...], preferred_element_type=F32)


def _main_kernel(x_ref, w_ref, p2_ref, u_ref, b_ref, o_ref):
    acc = jnp.dot(x_ref[...], w_ref[...], preferred_element_type=F32)
    adapt = jax.lax.dot_general(
        p2_ref[...], u_ref[...], (((1,), (1,)), ((), ())),
        preferred_element_type=F32)
    o_ref[...] = acc + adapt + b_ref[...]


@functools.partial(jax.jit, static_argnames=())
def kernel(x, tensor_idx, expert_weights, W, b, G, C, U):
    n, d_in = x.shape
    d_out = W.shape[1]
    k_dim, p_dim, r, _ = G.shape
    t_dim = expert_weights.shape[0]

    idx = jnp.asarray(tensor_idx, jnp.int32).reshape((1,))
    ew2 = expert_weights.reshape(1, t_dim).astype(F32)

    m_eff = pl.pallas_call(
        _prep_kernel,
        grid_spec=pltpu.PrefetchScalarGridSpec(
            num_scalar_prefetch=1,
            grid=(1,),
            in_specs=[
                pl.BlockSpec((1, t_dim), lambda i, idx_ref: (0, 0)),
                pl.BlockSpec((t_dim, p_dim), lambda i, idx_ref: (0, 0)),
                pl.BlockSpec((k_dim, p_dim, r, r), lambda i, idx_ref: (0, 0, 0, 0)),
                pl.BlockSpec((d_in, r), lambda i, idx_ref: (0, 0)),
            ],
            out_specs=pl.BlockSpec((r, r), lambda i, idx_ref: (0, 0)),
        ),
        out_shape=jax.ShapeDtypeStruct((r, r), F32),
    )(idx, ew2, C, G, U)

    xb = x.astype(jnp.bfloat16)
    wb = W.astype(jnp.bfloat16)
    ub = U.astype(jnp.bfloat16)

    bn2 = 1024
    p2 = pl.pallas_call(
        _p2_kernel,
        grid=(n // bn2,),
        in_specs=[
            pl.BlockSpec((bn2, d_in), lambda i: (i, 0)),
            pl.BlockSpec((d_in, r), lambda i: (0, 0)),
            pl.BlockSpec((r, r), lambda i: (0, 0)),
        ],
        out_specs=pl.BlockSpec((bn2, r), lambda i: (i, 0)),
        out_shape=jax.ShapeDtypeStruct((n, r), F32),
        compiler_params=pltpu.CompilerParams(
            dimension_semantics=("parallel",)),
    )(xb, ub, m_eff)

    bn, bj = 512, 512
    b2 = b.reshape(1, d_out)
    out = pl.pallas_call(
        _main_kernel,
        grid=(n // bn, d_out // bj),
        in_specs=[
            pl.BlockSpec((bn, d_in), lambda i, j: (i, 0)),
            pl.BlockSpec((d_in, bj), lambda i, j: (0, j)),
            pl.BlockSpec((bn, r), lambda i, j: (i, 0)),
            pl.BlockSpec((bj, r), lambda i, j: (j, 0)),
            pl.BlockSpec((1, bj), lambda i, j: (0, j)),
        ],
        out_specs=pl.BlockSpec((bn, bj), lambda i, j: (i, j)),
        out_shape=jax.ShapeDtypeStruct((n, d_out), F32),
        compiler_params=pltpu.CompilerParams(
            dimension_semantics=("parallel", "parallel")),
    )(xb, wb, p2, U, b2)
    return out


# speedup vs baseline: 1.6280x; 1.5073x over previous
import jax
import jax.numpy as jnp
from jax.experimental import pallas as pl
from jax.experimental.pallas import tpu as pltpu

F32 = jnp.float32
BF16 = jnp.bfloat16


def _prep_kernel(idx_ref, ew_ref, c_ref, g_ref, u_ref, m_ref):
    idx = idx_ref[0]
    g = g_ref[idx]
    c = c_ref[...]
    ew = ew_ref[...]
    w = jnp.dot(ew, c, preferred_element_type=F32)
    p_dim, r, _ = g.shape
    m0 = jnp.zeros((r, r), dtype=F32)
    for p in range(p_dim):
        onehot = (jax.lax.broadcasted_iota(jnp.int32, (p_dim, 1), 0) == p)
        wp = jnp.dot(w, onehot.astype(F32), preferred_element_type=F32)
        m0 = m0 + wp * g[p]
    gn2 = jnp.sum(g * g)
    cn2 = jnp.sum(c * c)
    un2 = jnp.sum(u_ref[...] * u_ref[...])
    scale = jax.lax.rsqrt(gn2) * jax.lax.rsqrt(cn2) / un2
    m_ref[...] = m0 * scale


def _weff_kernel(w_ref, ui_ref, uall_ref, m_ref, o_ref):
    a = jnp.dot(ui_ref[...], m_ref[...], preferred_element_type=F32)
    adapt = jax.lax.dot_general(
        a, uall_ref[...], (((1,), (1,)), ((), ())),
        preferred_element_type=F32)
    o_ref[...] = (w_ref[...] + adapt).astype(BF16)


def _main_kernel(x_ref, w_ref, b_ref, o_ref):
    xb = x_ref[...].astype(BF16)
    o_ref[...] = (jnp.dot(xb, w_ref[...], preferred_element_type=F32)
                  + b_ref[...])


def kernel(x, tensor_idx, expert_weights, W, b, G, C, U):
    n, d_in = x.shape
    d_out = W.shape[1]
    k_dim, p_dim, r, _ = G.shape
    t_dim = expert_weights.shape[0]

    idx = jnp.asarray(tensor_idx, jnp.int32).reshape((1,))
    ew2 = expert_weights.reshape(1, t_dim).astype(F32)

    m_eff = pl.pallas_call(
        _prep_kernel,
        grid_spec=pltpu.PrefetchScalarGridSpec(
            num_scalar_prefetch=1,
            grid=(1,),
            in_specs=[
                pl.BlockSpec((1, t_dim), lambda i, idx_ref: (0, 0)),
                pl.BlockSpec((t_dim, p_dim), lambda i, idx_ref: (0, 0)),
                pl.BlockSpec((k_dim, p_dim, r, r), lambda i, idx_ref: (0, 0, 0, 0)),
                pl.BlockSpec((d_in, r), lambda i, idx_ref: (0, 0)),
            ],
            out_specs=pl.BlockSpec((r, r), lambda i, idx_ref: (0, 0)),
        ),
        out_shape=jax.ShapeDtypeStruct((r, r), F32),
    )(idx, ew2, C, G, U)

    bw = 1024
    w_eff = pl.pallas_call(
        _weff_kernel,
        grid=(d_in // bw,),
        in_specs=[
            pl.BlockSpec((bw, d_out), lambda i: (i, 0)),
            pl.BlockSpec((bw, r), lambda i: (i, 0)),
            pl.BlockSpec((d_out, r), lambda i: (0, 0)),
            pl.BlockSpec((r, r), lambda i: (0, 0)),
        ],
        out_specs=pl.BlockSpec((bw, d_out), lambda i: (i, 0)),
        out_shape=jax.ShapeDtypeStruct((d_in, d_out), BF16),
        compiler_params=pltpu.CompilerParams(
            dimension_semantics=("parallel",)),
    )(W, U, U, m_eff)

    bn = 256
    b2 = b.reshape(1, d_out)
    out = pl.pallas_call(
        _main_kernel,
        grid=(n // bn,),
        in_specs=[
            pl.BlockSpec((bn, d_in), lambda i: (i, 0)),
            pl.BlockSpec((d_in, d_out), lambda i: (0, 0)),
            pl.BlockSpec((1, d_out), lambda i: (0, 0)),
        ],
        out_specs=pl.BlockSpec((bn, d_out), lambda i: (i, 0)),
        out_shape=jax.ShapeDtypeStruct((n, d_out), F32),
        compiler_params=pltpu.CompilerParams(
            dimension_semantics=("parallel",)),
    )(x, w_eff, b2)
    return out
